# baseline (speedup 1.0000x reference)
import functools

import jax
import jax.numpy as jnp
from jax import lax
from jax.experimental import pallas as pl
from jax.experimental.pallas import tpu as pltpu
from jax.experimental.pallas import tpu_sc as plsc

_DELTA_T = 30.0
_NODE_STD = 2.0
_EDGE_STD = 1.5
_NUM_GRAPHS = 64
_NW = 32
_LANES = 16
_NSLAB = 2

_params = pltpu.CompilerParams(needs_layout_passes=False)


def _wid():
    return lax.axis_index("s") * 2 + lax.axis_index("c")


@functools.lru_cache(maxsize=None)
def _build(n_nodes, n_edges, interpret=False):
    _mesh = plsc.VectorSubcoreMesh(
        core_axis_name="c", subcore_axis_name="s",
        num_cores=2, num_subcores=16)
    nslab = _NSLAB if n_edges % (_NSLAB * _NW) == 0 else 1
    eps = n_edges // nslab
    ept = eps // _NW
    chunk = 2000 if ept % 2000 == 0 else ept
    assert ept % chunk == 0 and chunk % 8 == 0
    nchunks = ept // chunk
    ngrp_e = chunk // _LANES

    nb = -(-n_nodes // (_NW * _LANES)) * _LANES
    last_base = n_nodes - nb
    assert last_base >= 0 and last_base % 8 == 0 and n_nodes % 8 == 0
    ngrp_n = nb // _LANES

    @functools.partial(
        pl.kernel,
        out_type=jax.ShapeDtypeStruct((_NW * n_nodes,), jnp.float32),
        mesh=_mesh,
        scratch_types=[
            pltpu.VMEM((n_nodes,), jnp.float32),
            pltpu.VMEM((chunk,), jnp.float32),
            pltpu.VMEM((chunk,), jnp.float32),
            pltpu.VMEM((chunk,), jnp.int32),
            pltpu.VMEM((chunk,), jnp.int32),
            pltpu.VMEM((chunk,), jnp.int32),
            pltpu.VMEM((chunk,), jnp.int32),
            pltpu.SemaphoreType.DMA,
            pltpu.SemaphoreType.DMA,
        ],
        compiler_params=_params,
        interpret=interpret,
    )
    def scatter_net(wf, row, col, out_hbm, acc,
                    fbuf0, fbuf1, rbuf0, rbuf1, cbuf0, cbuf1, sem0, sem1):
        wid = _wid()
        bufs = ((fbuf0, rbuf0, cbuf0, sem0), (fbuf1, rbuf1, cbuf1, sem1))

        def zinit(i, _):
            acc[pl.ds(i * _LANES, _LANES)] = jnp.zeros((_LANES,), jnp.float32)
            return 0

        lax.fori_loop(0, n_nodes // _LANES, zinit, 0)

        ebase = wid * ept

        def start(j, slot):
            fb, rb, cb, sem = bufs[slot]
            base = ebase + j * chunk
            pltpu.async_copy(wf.at[pl.ds(base, chunk)], fb, sem)
            pltpu.async_copy(row.at[pl.ds(base, chunk)], rb, sem)
            pltpu.async_copy(col.at[pl.ds(base, chunk)], cb, sem)

        def wait(j, slot):
            fb, rb, cb, sem = bufs[slot]
            base = ebase + j * chunk
            pltpu.make_async_copy(wf.at[pl.ds(base, chunk)], fb, sem).wait()
            pltpu.make_async_copy(row.at[pl.ds(base, chunk)], rb, sem).wait()
            pltpu.make_async_copy(col.at[pl.ds(base, chunk)], cb, sem).wait()

        def process(j, slot):
            fb, rb, cb, _ = bufs[slot]

            @pl.when(j + 1 < nchunks)
            def _():
                start(j + 1, 1 - slot)

            wait(j, slot)

            def grp(g, _):
                sl = pl.ds(g * _LANES, _LANES)
                f = fb[sl]
                r = rb[sl]
                c = cb[sl]
                v = f * (_EDGE_STD * _DELTA_T)
                plsc.addupdate_scatter(acc, [c], v)
                plsc.addupdate_scatter(acc, [r], -v)
                return 0

            lax.fori_loop(0, ngrp_e, grp, 0)

        start(0, 0)

        def chunk_pair(jj, _):
            j0 = jj * 2
            process(j0, 0)

            @pl.when(j0 + 1 < nchunks)
            def _():
                process(j0 + 1, 1)

            return 0

        lax.fori_loop(0, (nchunks + 1) // 2, chunk_pair, 0)
        pltpu.sync_copy(acc, out_hbm.at[pl.ds(wid * n_nodes, n_nodes)])

    @functools.partial(
        pl.kernel,
        out_type=jax.ShapeDtypeStruct((_NW * _LANES,), jnp.float32),
        mesh=_mesh,
        scratch_types=[
            pltpu.VMEM((nb,), jnp.float32),
            pltpu.VMEM((nb,), jnp.float32),
            pltpu.VMEM((nb,), jnp.float32),
            pltpu.VMEM((nb,), jnp.float32),
            pltpu.VMEM((nb,), jnp.float32),
            pltpu.VMEM((nb,), jnp.float32),
            pltpu.VMEM((nb,), jnp.float32),
            pltpu.VMEM((_LANES,), jnp.float32),
            pltpu.SemaphoreType.DMA,
            pltpu.SemaphoreType.DMA,
        ],
        compiler_params=_params,
        interpret=interpret,
    )
    def node_loss(*args):
        parts_list = args[:nslab]
        cn_h, pn_h, rain, maskf, out_hbm = args[nslab:nslab + 5]
        (net, stage0, stage1, ncn, npn, nrf, nmk, pout,
         sem0, sem1) = args[nslab + 5:]
        wid = _wid()
        stages = ((stage0, sem0), (stage1, sem1))
        iota = lax.iota(jnp.int32, _LANES)
        base = jnp.minimum(wid * nb, last_base)

        pltpu.sync_copy(cn_h.at[pl.ds(base, nb)], ncn)
        pltpu.sync_copy(pn_h.at[pl.ds(base, nb)], npn)
        pltpu.sync_copy(rain.at[pl.ds(base, nb)], nrf)
        pltpu.sync_copy(maskf.at[pl.ds(base, nb)], nmk)
        pltpu.sync_copy(parts_list[0].at[pl.ds(base, nb)], net)

        def pstart(parts, j, slot):
            st, sem = stages[slot]
            pltpu.async_copy(parts.at[pl.ds(j * n_nodes + base, nb)], st, sem)

        def pprocess(parts, j, slot, next_parts, jn):
            st, sem = stages[slot]

            if next_parts is not None:
                @pl.when(jn < _NW)
                def _():
                    pstart(next_parts, jn, 1 - slot)

            pltpu.make_async_copy(parts.at[pl.ds(j * n_nodes + base, nb)],
                                  st, sem).wait()

            def add_grp(g, _):
                sl = pl.ds(g * _LANES, _LANES)
                net[sl] = net[sl] + st[sl]
                return 0

            lax.fori_loop(0, ngrp_n, add_grp, 0)

        pstart(parts_list[0], 1, 1)

        def slab_rows(parts, start_row):
            def pair(jj, _):
                j1 = start_row + jj * 2

                @pl.when(j1 < _NW)
                def _():
                    pprocess(parts, j1, 1, parts, j1 + 1)

                @pl.when(j1 + 1 < _NW)
                def _():
                    pprocess(parts, j1 + 1, 0, parts, j1 + 2)

                return 0
            return pair

        lax.fori_loop(0, _NW // 2, slab_rows(parts_list[0], 1), 0)
        for s in range(1, nslab):
            pstart(parts_list[s], 0, 1)
            lax.fori_loop(0, _NW // 2, slab_rows(parts_list[s], 0), 0)

        lo_valid = wid * nb

        def grp(g, carry):
            sl = pl.ds(g * _LANES, _LANES)
            dv = (npn[sl] - ncn[sl]) * _NODE_STD
            e = dv - net[sl] - nrf[sl]
            err = jnp.abs(e) * nmk[sl]
            gidx = base + g * _LANES + iota
            ok = jnp.logical_and(gidx >= lo_valid, gidx < n_nodes)
            return carry + jnp.where(ok, err, jnp.zeros_like(err))

        partial = lax.fori_loop(0, ngrp_n, grp, jnp.zeros((_LANES,), jnp.float32))
        pout[...] = partial
        pltpu.sync_copy(pout, out_hbm.at[pl.ds(wid * _LANES, _LANES)])

    def run(batch_node_pred, batch_node_input, batch_edge_input, batch,
            edge_index, rainfall, non_boundary_nodes_mask):
        del batch
        ei = edge_index.astype(jnp.int32)
        cn = batch_node_input[:, 9]
        pn = batch_node_pred[:, 1]
        maskf = non_boundary_nodes_mask.astype(jnp.float32)
        parts = []
        for s in range(nslab):
            sl = slice(s * eps, (s + 1) * eps)
            wf_s = batch_edge_input[sl, 2]
            row_s = ei[0, sl]
            col_s = ei[1, sl]
            parts.append(scatter_net(wf_s, row_s, col_s))
        pt = node_loss(*parts, cn, pn, rainfall, maskf)
        return jnp.sum(pt) / _NUM_GRAPHS

    return jax.jit(run)


def kernel(batch_node_pred, batch_node_input, batch_edge_input, batch,
           edge_index, rainfall, non_boundary_nodes_mask):
    n_nodes = batch_node_input.shape[0]
    n_edges = batch_edge_input.shape[0]
    fn = _build(n_nodes, n_edges)
    return fn(batch_node_pred, batch_node_input, batch_edge_input, batch,
              edge_index, rainfall, non_boundary_nodes_mask)


# speedup vs baseline: 1.3133x; 1.3133x over previous
import functools

import jax
import jax.numpy as jnp
from jax import lax
from jax.experimental import pallas as pl
from jax.experimental.pallas import tpu as pltpu
from jax.experimental.pallas import tpu_sc as plsc

_DELTA_T = 30.0
_NODE_STD = 2.0
_EDGE_STD = 1.5
_NUM_GRAPHS = 64
_NW = 32
_LANES = 16

_params = pltpu.CompilerParams(needs_layout_passes=False)


def _wid():
    return lax.axis_index("s") * 2 + lax.axis_index("c")


@functools.lru_cache(maxsize=None)
def _build(n_nodes, n_edges, interpret=False):
    _mesh = plsc.VectorSubcoreMesh(
        core_axis_name="c", subcore_axis_name="s",
        num_cores=2, num_subcores=16)
    assert n_edges % _NW == 0
    ept = n_edges // _NW
    chunk = 2000 if ept % 2000 == 0 else ept
    assert ept % chunk == 0 and chunk % 8 == 0
    nchunks = ept // chunk
    ngrp_e = chunk // _LANES

    nb = -(-n_nodes // (_NW * _LANES)) * _LANES
    last_base = n_nodes - nb
    assert last_base >= 0 and last_base % 8 == 0 and n_nodes % 8 == 0
    ngrp_n = nb // _LANES

    @functools.partial(
        pl.kernel,
        out_type=jax.ShapeDtypeStruct((_NW * n_nodes,), jnp.float32),
        mesh=_mesh,
        scratch_types=[
            pltpu.VMEM((n_nodes,), jnp.float32),
            pltpu.VMEM((chunk,), jnp.float32),
            pltpu.VMEM((chunk,), jnp.float32),
            pltpu.VMEM((chunk,), jnp.int32),
            pltpu.VMEM((chunk,), jnp.int32),
            pltpu.VMEM((chunk,), jnp.int32),
            pltpu.VMEM((chunk,), jnp.int32),
            pltpu.SemaphoreType.DMA,
            pltpu.SemaphoreType.DMA,
        ],
        compiler_params=_params,
        interpret=interpret,
    )
    def scatter_net(wf, ei, out_hbm, acc,
                    fbuf0, fbuf1, rbuf0, rbuf1, cbuf0, cbuf1, sem0, sem1):
        wid = _wid()
        bufs = ((fbuf0, rbuf0, cbuf0, sem0), (fbuf1, rbuf1, cbuf1, sem1))

        def zinit(i, _):
            acc[pl.ds(i * _LANES, _LANES)] = jnp.zeros((_LANES,), jnp.float32)
            return 0

        lax.fori_loop(0, n_nodes // _LANES, zinit, 0)

        ebase = wid * ept

        def start(j, slot):
            fb, rb, cb, sem = bufs[slot]
            base = ebase + j * chunk
            pltpu.async_copy(wf.at[pl.ds(base, chunk)], fb, sem)
            pltpu.async_copy(ei.at[pl.ds(base, chunk)], rb, sem)
            pltpu.async_copy(ei.at[pl.ds(n_edges + base, chunk)], cb, sem)

        def wait(j, slot):
            fb, rb, cb, sem = bufs[slot]
            base = ebase + j * chunk
            pltpu.make_async_copy(wf.at[pl.ds(base, chunk)], fb, sem).wait()
            pltpu.make_async_copy(ei.at[pl.ds(base, chunk)], rb, sem).wait()
            pltpu.make_async_copy(ei.at[pl.ds(n_edges + base, chunk)], cb,
                                  sem).wait()

        def process(j, slot):
            fb, rb, cb, _ = bufs[slot]

            @pl.when(j + 1 < nchunks)
            def _():
                start(j + 1, 1 - slot)

            wait(j, slot)

            def grp(g, _):
                sl = pl.ds(g * _LANES, _LANES)
                f = fb[sl]
                r = rb[sl]
                c = cb[sl]
                v = f * (_EDGE_STD * _DELTA_T)
                plsc.addupdate_scatter(acc, [c], v)
                plsc.addupdate_scatter(acc, [r], -v)
                return 0

            lax.fori_loop(0, ngrp_e, grp, 0)

        start(0, 0)

        def chunk_pair(jj, _):
            j0 = jj * 2
            process(j0, 0)

            @pl.when(j0 + 1 < nchunks)
            def _():
                process(j0 + 1, 1)

            return 0

        lax.fori_loop(0, (nchunks + 1) // 2, chunk_pair, 0)
        pltpu.sync_copy(acc, out_hbm.at[pl.ds(wid * n_nodes, n_nodes)])

    @functools.partial(
        pl.kernel,
        out_type=jax.ShapeDtypeStruct((_NW * _LANES,), jnp.float32),
        mesh=_mesh,
        scratch_types=[
            pltpu.VMEM((nb,), jnp.float32),
            pltpu.VMEM((nb,), jnp.float32),
            pltpu.VMEM((nb,), jnp.float32),
            pltpu.VMEM((nb,), jnp.float32),
            pltpu.VMEM((nb,), jnp.float32),
            pltpu.VMEM((nb,), jnp.float32),
            pltpu.VMEM((nb,), jnp.float32),
            pltpu.VMEM((_LANES,), jnp.float32),
            pltpu.SemaphoreType.DMA,
            pltpu.SemaphoreType.DMA,
        ],
        compiler_params=_params,
        interpret=interpret,
    )
    def node_loss(parts, cn_h, pn_h, rain, maskf, out_hbm,
                  net, stage0, stage1, ncn, npn, nrf, nmk, pout, sem0, sem1):
        wid = _wid()
        stages = ((stage0, sem0), (stage1, sem1))
        iota = lax.iota(jnp.int32, _LANES)
        base = jnp.minimum(wid * nb, last_base)

        pltpu.sync_copy(cn_h.at[pl.ds(base, nb)], ncn)
        pltpu.sync_copy(pn_h.at[pl.ds(base, nb)], npn)
        pltpu.sync_copy(rain.at[pl.ds(base, nb)], nrf)
        pltpu.sync_copy(maskf.at[pl.ds(base, nb)], nmk)
        pltpu.sync_copy(parts.at[pl.ds(base, nb)], net)

        def pstart(j, slot):
            st, sem = stages[slot]
            pltpu.async_copy(parts.at[pl.ds(j * n_nodes + base, nb)], st, sem)

        def pprocess(j, slot):
            st, sem = stages[slot]

            @pl.when(j + 1 < _NW)
            def _():
                pstart(j + 1, 1 - slot)

            pltpu.make_async_copy(parts.at[pl.ds(j * n_nodes + base, nb)],
                                  st, sem).wait()

            def add_grp(g, _):
                sl = pl.ds(g * _LANES, _LANES)
                net[sl] = net[sl] + st[sl]
                return 0

            lax.fori_loop(0, ngrp_n, add_grp, 0)

        pstart(1, 1)

        def part_pair(jj, _):
            j1 = jj * 2 + 1

            @pl.when(j1 < _NW)
            def _():
                pprocess(j1, 1)

            @pl.when(j1 + 1 < _NW)
            def _():
                pprocess(j1 + 1, 0)

            return 0

        lax.fori_loop(0, _NW // 2, part_pair, 0)

        lo_valid = wid * nb

        def grp(g, carry):
            sl = pl.ds(g * _LANES, _LANES)
            dv = (npn[sl] - ncn[sl]) * _NODE_STD
            e = dv - net[sl] - nrf[sl]
            err = jnp.abs(e) * nmk[sl]
            gidx = base + g * _LANES + iota
            ok = jnp.logical_and(gidx >= lo_valid, gidx < n_nodes)
            return carry + jnp.where(ok, err, jnp.zeros_like(err))

        partial = lax.fori_loop(0, ngrp_n, grp, jnp.zeros((_LANES,), jnp.float32))
        pout[...] = partial
        pltpu.sync_copy(pout, out_hbm.at[pl.ds(wid * _LANES, _LANES)])

    def run(batch_node_pred, batch_node_input, batch_edge_input, batch,
            edge_index, rainfall, non_boundary_nodes_mask):
        del batch
        wf = batch_edge_input[:, 2]
        ei = edge_index.astype(jnp.int32).reshape(-1)
        cn = batch_node_input[:, 9]
        pn = batch_node_pred[:, 1]
        maskf = non_boundary_nodes_mask.astype(jnp.float32)
        parts = scatter_net(wf, ei)
        pt = node_loss(parts, cn, pn, rainfall, maskf)
        return jnp.sum(pt) / _NUM_GRAPHS

    return jax.jit(run)


def kernel(batch_node_pred, batch_node_input, batch_edge_input, batch,
           edge_index, rainfall, non_boundary_nodes_mask):
    n_nodes = batch_node_input.shape[0]
    n_edges = batch_edge_input.shape[0]
    fn = _build(n_nodes, n_edges)
    return fn(batch_node_pred, batch_node_input, batch_edge_input, batch,
              edge_index, rainfall, non_boundary_nodes_mask)


# speedup vs baseline: 1.5006x; 1.1426x over previous
import functools

import jax
import jax.numpy as jnp
from jax import lax
from jax.experimental import pallas as pl
from jax.experimental.pallas import tpu as pltpu
from jax.experimental.pallas import tpu_sc as plsc

_DELTA_T = 30.0
_NODE_STD = 2.0
_EDGE_STD = 1.5
_NUM_GRAPHS = 64
_NW = 32
_LANES = 16

_params = pltpu.CompilerParams(needs_layout_passes=False)


def _wid():
    return lax.axis_index("s") * 2 + lax.axis_index("c")


@functools.lru_cache(maxsize=None)
def _build(n_nodes, n_edges, interpret=False):
    _mesh = plsc.VectorSubcoreMesh(
        core_axis_name="c", subcore_axis_name="s",
        num_cores=2, num_subcores=16)
    nb = -(-n_nodes // (_NW * _LANES)) * _LANES
    last_base = n_nodes - nb
    assert last_base >= 0 and last_base % 8 == 0 and n_nodes % 8 == 0
    ngrp_n = nb // _LANES

    nt = n_edges // 128
    assert n_edges % (128 * _LANES) == 0
    nchk = nt // _LANES
    npw = -(-nchk // _NW)
    echunk = 128 * _LANES

    @functools.partial(
        pl.kernel,
        out_type=jax.ShapeDtypeStruct((_NW * n_nodes,), jnp.float32),
        mesh=_mesh,
        scratch_types=[
            pltpu.VMEM((n_nodes,), jnp.float32),
            pltpu.VMEM((echunk,), jnp.float32),
            pltpu.VMEM((echunk,), jnp.float32),
            pltpu.VMEM((2 * _LANES, 128), jnp.int32),
            pltpu.VMEM((2 * _LANES, 128), jnp.int32),
            pltpu.SemaphoreType.DMA,
            pltpu.SemaphoreType.DMA,
        ],
        compiler_params=_params,
        interpret=interpret,
    )
    def scatter_net(wf, eiv, out_hbm, acc,
                    fbuf0, fbuf1, ibuf0, ibuf1, sem0, sem1):
        wid = _wid()
        bufs = ((fbuf0, ibuf0, sem0), (fbuf1, ibuf1, sem1))

        def zinit(i, _):
            acc[pl.ds(i * _LANES, _LANES)] = jnp.zeros((_LANES,), jnp.float32)
            return 0

        lax.fori_loop(0, n_nodes // _LANES, zinit, 0)

        def start(j, slot):
            fb, ib, sem = bufs[slot]
            cid = wid + _NW * j
            pltpu.async_copy(wf.at[pl.ds(cid * echunk, echunk)], fb, sem)
            pltpu.async_copy(eiv.at[pl.ds(cid * (2 * _LANES), 2 * _LANES), :],
                             ib, sem)

        def wait(j, slot):
            fb, ib, sem = bufs[slot]
            cid = wid + _NW * j
            pltpu.make_async_copy(wf.at[pl.ds(cid * echunk, echunk)], fb,
                                  sem).wait()
            pltpu.make_async_copy(
                eiv.at[pl.ds(cid * (2 * _LANES), 2 * _LANES), :], ib,
                sem).wait()

        def process(j, slot):
            fb, ib, _ = bufs[slot]
            cid = wid + _NW * j

            @pl.when(cid + _NW < nchk)
            def _():
                start(j + 1, 1 - slot)

            wait(j, slot)

            def tloop(t, _):
                def grp(g, _):
                    sl = pl.ds(g * _LANES, _LANES)
                    f = fb[pl.ds(t * 128 + g * _LANES, _LANES)]
                    r = ib[2 * t, sl]
                    c = ib[2 * t + 1, sl]
                    v = f * (_EDGE_STD * _DELTA_T)
                    plsc.addupdate_scatter(acc, [c], v)
                    plsc.addupdate_scatter(acc, [r], -v)
                    return 0

                lax.fori_loop(0, 128 // _LANES, grp, 0)
                return 0

            lax.fori_loop(0, _LANES, tloop, 0)

        start(0, 0)

        def chunk_pair(jj, _):
            j0 = jj * 2

            @pl.when(wid + _NW * j0 < nchk)
            def _():
                process(j0, 0)

            @pl.when(wid + _NW * (j0 + 1) < nchk)
            def _():
                process(j0 + 1, 1)

            return 0

        lax.fori_loop(0, (npw + 1) // 2, chunk_pair, 0)
        pltpu.sync_copy(acc, out_hbm.at[pl.ds(wid * n_nodes, n_nodes)])

    @functools.partial(
        pl.kernel,
        out_type=jax.ShapeDtypeStruct((_NW * _LANES,), jnp.float32),
        mesh=_mesh,
        scratch_types=[
            pltpu.VMEM((nb,), jnp.float32),
            pltpu.VMEM((nb,), jnp.float32),
            pltpu.VMEM((nb,), jnp.float32),
            pltpu.VMEM((nb,), jnp.float32),
            pltpu.VMEM((nb,), jnp.float32),
            pltpu.VMEM((nb,), jnp.float32),
            pltpu.VMEM((nb,), jnp.float32),
            pltpu.VMEM((_LANES,), jnp.float32),
            pltpu.SemaphoreType.DMA,
            pltpu.SemaphoreType.DMA,
        ],
        compiler_params=_params,
        interpret=interpret,
    )
    def node_loss(parts, cn_h, pn_h, rain, maskf, out_hbm,
                  net, stage0, stage1, ncn, npn, nrf, nmk, pout, sem0, sem1):
        wid = _wid()
        stages = ((stage0, sem0), (stage1, sem1))
        iota = lax.iota(jnp.int32, _LANES)
        base = jnp.minimum(wid * nb, last_base)

        pltpu.sync_copy(cn_h.at[pl.ds(base, nb)], ncn)
        pltpu.sync_copy(pn_h.at[pl.ds(base, nb)], npn)
        pltpu.sync_copy(rain.at[pl.ds(base, nb)], nrf)
        pltpu.sync_copy(maskf.at[pl.ds(base, nb)], nmk)
        pltpu.sync_copy(parts.at[pl.ds(base, nb)], net)

        def pstart(j, slot):
            st, sem = stages[slot]
            pltpu.async_copy(parts.at[pl.ds(j * n_nodes + base, nb)], st, sem)

        def pprocess(j, slot):
            st, sem = stages[slot]

            @pl.when(j + 1 < _NW)
            def _():
                pstart(j + 1, 1 - slot)

            pltpu.make_async_copy(parts.at[pl.ds(j * n_nodes + base, nb)],
                                  st, sem).wait()

            def add_grp(g, _):
                sl = pl.ds(g * _LANES, _LANES)
                net[sl] = net[sl] + st[sl]
                return 0

            lax.fori_loop(0, ngrp_n, add_grp, 0)

        pstart(1, 1)

        def part_pair(jj, _):
            j1 = jj * 2 + 1

            @pl.when(j1 < _NW)
            def _():
                pprocess(j1, 1)

            @pl.when(j1 + 1 < _NW)
            def _():
                pprocess(j1 + 1, 0)

            return 0

        lax.fori_loop(0, _NW // 2, part_pair, 0)

        lo_valid = wid * nb

        def grp(g, carry):
            sl = pl.ds(g * _LANES, _LANES)
            dv = (npn[sl] - ncn[sl]) * _NODE_STD
            e = dv - net[sl] - nrf[sl]
            err = jnp.abs(e) * nmk[sl]
            gidx = base + g * _LANES + iota
            ok = jnp.logical_and(gidx >= lo_valid, gidx < n_nodes)
            return carry + jnp.where(ok, err, jnp.zeros_like(err))

        partial = lax.fori_loop(0, ngrp_n, grp, jnp.zeros((_LANES,), jnp.float32))
        pout[...] = partial
        pltpu.sync_copy(pout, out_hbm.at[pl.ds(wid * _LANES, _LANES)])

    def run(batch_node_pred, batch_node_input, batch_edge_input, batch,
            edge_index, rainfall, non_boundary_nodes_mask):
        del batch
        wf = batch_edge_input[:, 2]
        ei32 = edge_index.astype(jnp.int32)
        eiv = jnp.transpose(ei32.reshape(2, n_edges // 128, 128),
                            (1, 0, 2)).reshape(n_edges // 64, 128)
        cn = batch_node_input[:, 9]
        pn = batch_node_pred[:, 1]
        maskf = non_boundary_nodes_mask.astype(jnp.float32)
        parts = scatter_net(wf, eiv)
        pt = node_loss(parts, cn, pn, rainfall, maskf)
        return jnp.sum(pt) / _NUM_GRAPHS

    return jax.jit(run)


def kernel(batch_node_pred, batch_node_input, batch_edge_input, batch,
           edge_index, rainfall, non_boundary_nodes_mask):
    n_nodes = batch_node_input.shape[0]
    n_edges = batch_edge_input.shape[0]
    fn = _build(n_nodes, n_edges)
    return fn(batch_node_pred, batch_node_input, batch_edge_input, batch,
              edge_index, rainfall, non_boundary_nodes_mask)


# speedup vs baseline: 1.7296x; 1.1526x over previous
import functools

import jax
import jax.numpy as jnp
from jax import lax
from jax.experimental import pallas as pl
from jax.experimental.pallas import tpu as pltpu
from jax.experimental.pallas import tpu_sc as plsc

_DELTA_T = 30.0
_NODE_STD = 2.0
_EDGE_STD = 1.5
_NUM_GRAPHS = 64
_NW = 32
_LANES = 16

_params = pltpu.CompilerParams(needs_layout_passes=False)


def _wid():
    return lax.axis_index("s") * 2 + lax.axis_index("c")


@functools.lru_cache(maxsize=None)
def _build(n_nodes, n_edges, interpret=False):
    _mesh = plsc.VectorSubcoreMesh(
        core_axis_name="c", subcore_axis_name="s",
        num_cores=2, num_subcores=16)
    nb = -(-n_nodes // (_NW * _LANES)) * _LANES
    last_base = n_nodes - nb
    assert last_base >= 0 and last_base % 8 == 0 and n_nodes % 8 == 0
    ngrp_n = nb // _LANES

    nt = n_edges // 128
    assert n_edges % (128 * _LANES) == 0
    nchk = nt // _LANES
    npw = -(-nchk // _NW)
    echunk = 128 * _LANES

    @functools.partial(
        pl.kernel,
        out_type=jax.ShapeDtypeStruct((_NW * n_nodes,), jnp.float32),
        mesh=_mesh,
        scratch_types=[
            pltpu.VMEM((n_nodes,), jnp.float32),
            pltpu.VMEM((echunk,), jnp.float32),
            pltpu.VMEM((echunk,), jnp.float32),
            pltpu.VMEM((2 * _LANES, 128), jnp.int32),
            pltpu.VMEM((2 * _LANES, 128), jnp.int32),
            pltpu.SemaphoreType.DMA,
            pltpu.SemaphoreType.DMA,
        ],
        compiler_params=_params,
        interpret=interpret,
    )
    def scatter_net(wf, eiv, out_hbm, acc,
                    fbuf0, fbuf1, ibuf0, ibuf1, sem0, sem1):
        wid = _wid()
        bufs = ((fbuf0, ibuf0, sem0), (fbuf1, ibuf1, sem1))

        def zinit(i, _):
            acc[pl.ds(i * _LANES, _LANES)] = jnp.zeros((_LANES,), jnp.float32)
            return 0

        lax.fori_loop(0, n_nodes // _LANES, zinit, 0)

        def start(j, slot):
            fb, ib, sem = bufs[slot]
            cid = wid + _NW * j
            pltpu.async_copy(wf.at[pl.ds(cid * echunk, echunk)], fb, sem)
            pltpu.async_copy(eiv.at[pl.ds(cid * (2 * _LANES), 2 * _LANES), :],
                             ib, sem)

        def wait(j, slot):
            fb, ib, sem = bufs[slot]
            cid = wid + _NW * j
            pltpu.make_async_copy(wf.at[pl.ds(cid * echunk, echunk)], fb,
                                  sem).wait()
            pltpu.make_async_copy(
                eiv.at[pl.ds(cid * (2 * _LANES), 2 * _LANES), :], ib,
                sem).wait()

        def process(j, slot):
            fb, ib, _ = bufs[slot]
            cid = wid + _NW * j

            @pl.when(cid + _NW < nchk)
            def _():
                start(j + 1, 1 - slot)

            wait(j, slot)

            @plsc.parallel_loop(0, echunk // _LANES, unroll=4)
            def _(gg):
                t = gg // 8
                g = gg - t * 8
                sl = pl.ds(g * _LANES, _LANES)
                f = fb[pl.ds(gg * _LANES, _LANES)]
                r = ib[2 * t, sl]
                c = ib[2 * t + 1, sl]
                v = f * (_EDGE_STD * _DELTA_T)
                plsc.addupdate_scatter(acc, [c], v)
                plsc.addupdate_scatter(acc, [r], -v)

        start(0, 0)

        def chunk_pair(jj, _):
            j0 = jj * 2

            @pl.when(wid + _NW * j0 < nchk)
            def _():
                process(j0, 0)

            @pl.when(wid + _NW * (j0 + 1) < nchk)
            def _():
                process(j0 + 1, 1)

            return 0

        lax.fori_loop(0, (npw + 1) // 2, chunk_pair, 0)
        pltpu.sync_copy(acc, out_hbm.at[pl.ds(wid * n_nodes, n_nodes)])

    @functools.partial(
        pl.kernel,
        out_type=jax.ShapeDtypeStruct((_NW * _LANES,), jnp.float32),
        mesh=_mesh,
        scratch_types=[
            pltpu.VMEM((nb,), jnp.float32),
            pltpu.VMEM((nb,), jnp.float32),
            pltpu.VMEM((nb,), jnp.float32),
            pltpu.VMEM((nb,), jnp.float32),
            pltpu.VMEM((nb,), jnp.float32),
            pltpu.VMEM((nb,), jnp.float32),
            pltpu.VMEM((nb,), jnp.float32),
            pltpu.VMEM((_LANES,), jnp.float32),
            pltpu.SemaphoreType.DMA,
            pltpu.SemaphoreType.DMA,
        ],
        compiler_params=_params,
        interpret=interpret,
    )
    def node_loss(parts, cn_h, pn_h, rain, maskf, out_hbm,
                  net, stage0, stage1, ncn, npn, nrf, nmk, pout, sem0, sem1):
        wid = _wid()
        stages = ((stage0, sem0), (stage1, sem1))
        iota = lax.iota(jnp.int32, _LANES)
        base = jnp.minimum(wid * nb, last_base)

        pltpu.sync_copy(cn_h.at[pl.ds(base, nb)], ncn)
        pltpu.sync_copy(pn_h.at[pl.ds(base, nb)], npn)
        pltpu.sync_copy(rain.at[pl.ds(base, nb)], nrf)
        pltpu.sync_copy(maskf.at[pl.ds(base, nb)], nmk)
        pltpu.sync_copy(parts.at[pl.ds(base, nb)], net)

        def pstart(j, slot):
            st, sem = stages[slot]
            pltpu.async_copy(parts.at[pl.ds(j * n_nodes + base, nb)], st, sem)

        def pprocess(j, slot):
            st, sem = stages[slot]

            @pl.when(j + 1 < _NW)
            def _():
                pstart(j + 1, 1 - slot)

            pltpu.make_async_copy(parts.at[pl.ds(j * n_nodes + base, nb)],
                                  st, sem).wait()

            def add_grp(g, _):
                sl = pl.ds(g * _LANES, _LANES)
                net[sl] = net[sl] + st[sl]
                return 0

            lax.fori_loop(0, ngrp_n, add_grp, 0)

        pstart(1, 1)

        def part_pair(jj, _):
            j1 = jj * 2 + 1

            @pl.when(j1 < _NW)
            def _():
                pprocess(j1, 1)

            @pl.when(j1 + 1 < _NW)
            def _():
                pprocess(j1 + 1, 0)

            return 0

        lax.fori_loop(0, _NW // 2, part_pair, 0)

        lo_valid = wid * nb

        def grp(g, carry):
            sl = pl.ds(g * _LANES, _LANES)
            dv = (npn[sl] - ncn[sl]) * _NODE_STD
            e = dv - net[sl] - nrf[sl]
            err = jnp.abs(e) * nmk[sl]
            gidx = base + g * _LANES + iota
            ok = jnp.logical_and(gidx >= lo_valid, gidx < n_nodes)
            return carry + jnp.where(ok, err, jnp.zeros_like(err))

        partial = lax.fori_loop(0, ngrp_n, grp, jnp.zeros((_LANES,), jnp.float32))
        pout[...] = partial
        pltpu.sync_copy(pout, out_hbm.at[pl.ds(wid * _LANES, _LANES)])

    def run(batch_node_pred, batch_node_input, batch_edge_input, batch,
            edge_index, rainfall, non_boundary_nodes_mask):
        del batch
        wf = batch_edge_input[:, 2]
        ei32 = edge_index.astype(jnp.int32)
        eiv = jnp.transpose(ei32.reshape(2, n_edges // 128, 128),
                            (1, 0, 2)).reshape(n_edges // 64, 128)
        cn = batch_node_input[:, 9]
        pn = batch_node_pred[:, 1]
        maskf = non_boundary_nodes_mask.astype(jnp.float32)
        parts = scatter_net(wf, eiv)
        pt = node_loss(parts, cn, pn, rainfall, maskf)
        return jnp.sum(pt) / _NUM_GRAPHS

    return jax.jit(run)


def kernel(batch_node_pred, batch_node_input, batch_edge_input, batch,
           edge_index, rainfall, non_boundary_nodes_mask):
    n_nodes = batch_node_input.shape[0]
    n_edges = batch_edge_input.shape[0]
    fn = _build(n_nodes, n_edges)
    return fn(batch_node_pred, batch_node_input, batch_edge_input, batch,
              edge_index, rainfall, non_boundary_nodes_mask)


# speedup vs baseline: 1.7311x; 1.0008x over previous
import functools

import jax
import jax.numpy as jnp
from jax import lax
from jax.experimental import pallas as pl
from jax.experimental.pallas import tpu as pltpu
from jax.experimental.pallas import tpu_sc as plsc

_DELTA_T = 30.0
_NODE_STD = 2.0
_EDGE_STD = 1.5
_NUM_GRAPHS = 64
_NW = 32
_LANES = 16

_params = pltpu.CompilerParams(needs_layout_passes=False)


def _wid():
    return lax.axis_index("s") * 2 + lax.axis_index("c")


@functools.lru_cache(maxsize=None)
def _build(n_nodes, n_edges, interpret=False):
    _mesh = plsc.VectorSubcoreMesh(
        core_axis_name="c", subcore_axis_name="s",
        num_cores=2, num_subcores=16)
    nb = -(-n_nodes // (_NW * _LANES)) * _LANES
    last_base = n_nodes - nb
    assert last_base >= 0 and last_base % 8 == 0 and n_nodes % 8 == 0
    ngrp_n = nb // _LANES

    nt = n_edges // 128
    assert n_edges % (128 * _LANES) == 0
    nchk = nt // _LANES
    npw = -(-nchk // _NW)
    echunk = 128 * _LANES

    @functools.partial(
        pl.kernel,
        out_type=jax.ShapeDtypeStruct((_NW * n_nodes,), jnp.float32),
        mesh=_mesh,
        scratch_types=[
            pltpu.VMEM((n_nodes,), jnp.float32),
            pltpu.VMEM((echunk,), jnp.float32),
            pltpu.VMEM((echunk,), jnp.float32),
            pltpu.VMEM((2 * _LANES, 128), jnp.int32),
            pltpu.VMEM((2 * _LANES, 128), jnp.int32),
            pltpu.SemaphoreType.DMA,
            pltpu.SemaphoreType.DMA,
        ],
        compiler_params=_params,
        interpret=interpret,
    )
    def scatter_net(wf, eiv, out_hbm, acc,
                    fbuf0, fbuf1, ibuf0, ibuf1, sem0, sem1):
        wid = _wid()
        bufs = ((fbuf0, ibuf0, sem0), (fbuf1, ibuf1, sem1))

        def zinit(i, _):
            acc[pl.ds(i * _LANES, _LANES)] = jnp.zeros((_LANES,), jnp.float32)
            return 0

        lax.fori_loop(0, n_nodes // _LANES, zinit, 0)

        def start(j, slot):
            fb, ib, sem = bufs[slot]
            cid = wid + _NW * j
            pltpu.async_copy(wf.at[pl.ds(cid * echunk, echunk)], fb, sem)
            pltpu.async_copy(eiv.at[pl.ds(cid * (2 * _LANES), 2 * _LANES), :],
                             ib, sem)

        def wait(j, slot):
            fb, ib, sem = bufs[slot]
            cid = wid + _NW * j
            pltpu.make_async_copy(wf.at[pl.ds(cid * echunk, echunk)], fb,
                                  sem).wait()
            pltpu.make_async_copy(
                eiv.at[pl.ds(cid * (2 * _LANES), 2 * _LANES), :], ib,
                sem).wait()

        def process(j, slot):
            fb, ib, _ = bufs[slot]
            cid = wid + _NW * j

            @pl.when(cid + _NW < nchk)
            def _():
                start(j + 1, 1 - slot)

            wait(j, slot)

            @plsc.parallel_loop(0, echunk // _LANES, unroll=8)
            def _(gg):
                t = gg // 8
                g = gg - t * 8
                sl = pl.ds(g * _LANES, _LANES)
                f = fb[pl.ds(gg * _LANES, _LANES)]
                r = ib[2 * t, sl]
                c = ib[2 * t + 1, sl]
                v = f * (_EDGE_STD * _DELTA_T)
                plsc.addupdate_scatter(acc, [c], v)
                plsc.addupdate_scatter(acc, [r], -v)

        start(0, 0)

        def chunk_pair(jj, _):
            j0 = jj * 2

            @pl.when(wid + _NW * j0 < nchk)
            def _():
                process(j0, 0)

            @pl.when(wid + _NW * (j0 + 1) < nchk)
            def _():
                process(j0 + 1, 1)

            return 0

        lax.fori_loop(0, (npw + 1) // 2, chunk_pair, 0)
        pltpu.sync_copy(acc, out_hbm.at[pl.ds(wid * n_nodes, n_nodes)])

    @functools.partial(
        pl.kernel,
        out_type=jax.ShapeDtypeStruct((_NW * _LANES,), jnp.float32),
        mesh=_mesh,
        scratch_types=[
            pltpu.VMEM((nb,), jnp.float32),
            pltpu.VMEM((nb,), jnp.float32),
            pltpu.VMEM((nb,), jnp.float32),
            pltpu.VMEM((nb,), jnp.float32),
            pltpu.VMEM((nb,), jnp.float32),
            pltpu.VMEM((nb,), jnp.float32),
            pltpu.VMEM((nb,), jnp.float32),
            pltpu.VMEM((_LANES,), jnp.float32),
            pltpu.SemaphoreType.DMA,
            pltpu.SemaphoreType.DMA,
        ],
        compiler_params=_params,
        interpret=interpret,
    )
    def node_loss(parts, cn_h, pn_h, rain, maskf, out_hbm,
                  net, stage0, stage1, ncn, npn, nrf, nmk, pout, sem0, sem1):
        wid = _wid()
        stages = ((stage0, sem0), (stage1, sem1))
        iota = lax.iota(jnp.int32, _LANES)
        base = jnp.minimum(wid * nb, last_base)

        pltpu.sync_copy(cn_h.at[pl.ds(base, nb)], ncn)
        pltpu.sync_copy(pn_h.at[pl.ds(base, nb)], npn)
        pltpu.sync_copy(rain.at[pl.ds(base, nb)], nrf)
        pltpu.sync_copy(maskf.at[pl.ds(base, nb)], nmk)
        pltpu.sync_copy(parts.at[pl.ds(base, nb)], net)

        def pstart(j, slot):
            st, sem = stages[slot]
            pltpu.async_copy(parts.at[pl.ds(j * n_nodes + base, nb)], st, sem)

        def pprocess(j, slot):
            st, sem = stages[slot]

            @pl.when(j + 1 < _NW)
            def _():
                pstart(j + 1, 1 - slot)

            pltpu.make_async_copy(parts.at[pl.ds(j * n_nodes + base, nb)],
                                  st, sem).wait()

            def add_grp(g, _):
                sl = pl.ds(g * _LANES, _LANES)
                net[sl] = net[sl] + st[sl]
                return 0

            lax.fori_loop(0, ngrp_n, add_grp, 0)

        pstart(1, 1)

        def part_pair(jj, _):
            j1 = jj * 2 + 1

            @pl.when(j1 < _NW)
            def _():
                pprocess(j1, 1)

            @pl.when(j1 + 1 < _NW)
            def _():
                pprocess(j1 + 1, 0)

            return 0

        lax.fori_loop(0, _NW // 2, part_pair, 0)

        lo_valid = wid * nb

        def grp(g, carry):
            sl = pl.ds(g * _LANES, _LANES)
            dv = (npn[sl] - ncn[sl]) * _NODE_STD
            e = dv - net[sl] - nrf[sl]
            err = jnp.abs(e) * nmk[sl]
            gidx = base + g * _LANES + iota
            ok = jnp.logical_and(gidx >= lo_valid, gidx < n_nodes)
            return carry + jnp.where(ok, err, jnp.zeros_like(err))

        partial = lax.fori_loop(0, ngrp_n, grp, jnp.zeros((_LANES,), jnp.float32))
        pout[...] = partial
        pltpu.sync_copy(pout, out_hbm.at[pl.ds(wid * _LANES, _LANES)])

    def run(batch_node_pred, batch_node_input, batch_edge_input, batch,
            edge_index, rainfall, non_boundary_nodes_mask):
        del batch
        wf = batch_edge_input[:, 2]
        ei32 = edge_index.astype(jnp.int32)
        eiv = jnp.transpose(ei32.reshape(2, n_edges // 128, 128),
                            (1, 0, 2)).reshape(n_edges // 64, 128)
        cn = batch_node_input[:, 9]
        pn = batch_node_pred[:, 1]
        maskf = non_boundary_nodes_mask.astype(jnp.float32)
        parts = scatter_net(wf, eiv)
        pt = node_loss(parts, cn, pn, rainfall, maskf)
        return jnp.sum(pt) / _NUM_GRAPHS

    return jax.jit(run)


def kernel(batch_node_pred, batch_node_input, batch_edge_input, batch,
           edge_index, rainfall, non_boundary_nodes_mask):
    n_nodes = batch_node_input.shape[0]
    n_edges = batch_edge_input.shape[0]
    fn = _build(n_nodes, n_edges)
    return fn(batch_node_pred, batch_node_input, batch_edge_input, batch,
              edge_index, rainfall, non_boundary_nodes_mask)
